# initial kernel scaffold (unmeasured)
import jax
import jax.numpy as jnp
from jax import lax
from jax.experimental import pallas as pl
from jax.experimental.pallas import tpu as pltpu

N_DEV = 16


def kernel(x, w_mat, scale_x, scale_w):
    m, _ = x.shape
    _, n = w_mat.shape
    m_chunk = m // N_DEV

    xb = x.astype(jnp.bfloat16)
    wb = w_mat.astype(jnp.bfloat16)
    sx = scale_x.reshape(1).astype(jnp.float32)
    sw = scale_w.reshape(1).astype(jnp.float32)

    def body(x_ref, w_ref, sx_ref, sw_ref, out_ref, comm_ref, send_sems, recv_sems):
        d = lax.axis_index("i")
        right = lax.rem(d + 1, N_DEV)

        w = w_ref[...]

        def local_chunk(c):
            xc = x_ref[pl.ds(c * m_chunk, m_chunk), :]
            return jnp.dot(xc, w, preferred_element_type=jnp.float32)

        comm_ref[N_DEV - 1] = local_chunk(lax.rem(d - 1 + N_DEV, N_DEV))

        for s in range(N_DEV - 1):
            src_slot = (N_DEV - 1) if s == 0 else s - 1
            rdma = pltpu.make_async_remote_copy(
                src_ref=comm_ref.at[src_slot],
                dst_ref=comm_ref.at[s],
                send_sem=send_sems.at[s],
                recv_sem=recv_sems.at[s],
                device_id=(right,),
                device_id_type=pl.DeviceIdType.MESH,
            )
            rdma.start()
            rdma.wait()

            c = lax.rem(d - s - 2 + 2 * N_DEV, N_DEV)
            acc = comm_ref[s] + local_chunk(c)
            if s < N_DEV - 2:
                comm_ref[s] = acc
            else:
                scale = sx_ref[0] * sw_ref[0]
                y = acc * scale
                z = jnp.clip(y, -60.0, 60.0)
                out_ref[...] = y / (1.0 + jnp.exp(-z))

    return pl.pallas_call(
        body,
        out_shape=jax.ShapeDtypeStruct((m_chunk, n), jnp.float32),
        in_specs=[
            pl.BlockSpec(memory_space=pltpu.VMEM),
            pl.BlockSpec(memory_space=pltpu.VMEM),
            pl.BlockSpec(memory_space=pltpu.SMEM),
            pl.BlockSpec(memory_space=pltpu.SMEM),
        ],
        out_specs=pl.BlockSpec(memory_space=pltpu.VMEM),
        scratch_shapes=[
            pltpu.VMEM((N_DEV, m_chunk, n), jnp.float32),
            pltpu.SemaphoreType.DMA((N_DEV - 1,)),
            pltpu.SemaphoreType.DMA((N_DEV - 1,)),
        ],
    )(xb, wb, sx, sw)


# baseline (device time: 218837 ns/iter reference)
import jax
import jax.numpy as jnp
from jax import lax
from jax.experimental import pallas as pl
from jax.experimental.pallas import tpu as pltpu

N_DEV = 16


def kernel(x, w_mat, scale_x, scale_w):
    m, _ = x.shape
    _, n = w_mat.shape
    m_chunk = m // N_DEV

    xb = x.astype(jnp.bfloat16)
    wb = w_mat.astype(jnp.bfloat16)
    sx = scale_x.reshape(1).astype(jnp.float32)
    sw = scale_w.reshape(1).astype(jnp.float32)

    def body(x_ref, w_ref, sx_ref, sw_ref, out_ref, comm_ref, send_sems, recv_sems):
        d = lax.axis_index("i")
        right = lax.rem(d + 1, N_DEV)

        w = w_ref[...]

        def local_chunk(c):
            xc = x_ref[pl.ds(c * m_chunk, m_chunk), :]
            return jnp.dot(xc, w, preferred_element_type=jnp.float32)

        comm_ref[N_DEV - 1] = local_chunk(lax.rem(d - 1 + N_DEV, N_DEV)).astype(
            jnp.bfloat16
        )

        for s in range(N_DEV - 1):
            src_slot = (N_DEV - 1) if s == 0 else s - 1
            rdma = pltpu.make_async_remote_copy(
                src_ref=comm_ref.at[src_slot],
                dst_ref=comm_ref.at[s],
                send_sem=send_sems.at[s],
                recv_sem=recv_sems.at[s],
                device_id=(right,),
                device_id_type=pl.DeviceIdType.MESH,
            )
            rdma.start()
            rdma.wait()

            c = lax.rem(d - s - 2 + 2 * N_DEV, N_DEV)
            acc = comm_ref[s].astype(jnp.float32) + local_chunk(c)
            if s < N_DEV - 2:
                comm_ref[s] = acc.astype(jnp.bfloat16)
            else:
                scale = sx_ref[0] * sw_ref[0]
                y = acc * scale
                z = jnp.clip(y, -60.0, 60.0)
                out_ref[...] = y / (1.0 + jnp.exp(-z))

    return pl.pallas_call(
        body,
        out_shape=jax.ShapeDtypeStruct((m_chunk, n), jnp.float32),
        in_specs=[
            pl.BlockSpec(memory_space=pltpu.VMEM),
            pl.BlockSpec(memory_space=pltpu.VMEM),
            pl.BlockSpec(memory_space=pltpu.SMEM),
            pl.BlockSpec(memory_space=pltpu.SMEM),
        ],
        out_specs=pl.BlockSpec(memory_space=pltpu.VMEM),
        scratch_shapes=[
            pltpu.VMEM((N_DEV, m_chunk, n), jnp.bfloat16),
            pltpu.SemaphoreType.DMA((N_DEV - 1,)),
            pltpu.SemaphoreType.DMA((N_DEV - 1,)),
        ],
    )(xb, wb, sx, sw)


# device time: 152441 ns/iter; 1.4356x vs baseline; 1.4356x over previous
import jax
import jax.numpy as jnp
from jax import lax
from jax.experimental import pallas as pl
from jax.experimental.pallas import tpu as pltpu

N_DEV = 16


def kernel(x, w_mat, scale_x, scale_w):
    m, _ = x.shape
    _, n = w_mat.shape
    m_chunk = m // N_DEV
    n_half = n // 2

    xb = x.astype(jnp.bfloat16)
    wb = w_mat.astype(jnp.bfloat16)
    sx = scale_x.reshape(1).astype(jnp.float32)
    sw = scale_w.reshape(1).astype(jnp.float32)

    def body(
        x_ref,
        w_ref,
        sx_ref,
        sw_ref,
        out_ref,
        comm_f,
        comm_b,
        send_f,
        recv_f,
        send_b,
        recv_b,
    ):
        d = lax.axis_index("i")
        right = lax.rem(d + 1, N_DEV)
        left = lax.rem(d - 1 + N_DEV, N_DEV)

        def chunk_f(c):
            xc = x_ref[pl.ds(c * m_chunk, m_chunk), :]
            return jnp.dot(
                xc, w_ref[:, :n_half], preferred_element_type=jnp.float32
            )

        def chunk_b(c):
            xc = x_ref[pl.ds(c * m_chunk, m_chunk), :]
            return jnp.dot(
                xc, w_ref[:, n_half:], preferred_element_type=jnp.float32
            )

        def md(v):
            return lax.rem(v + 2 * N_DEV, N_DEV)

        comm_f[N_DEV - 1] = chunk_f(md(d - 1)).astype(jnp.bfloat16)
        comm_b[N_DEV - 1] = chunk_b(md(d + 1)).astype(jnp.bfloat16)

        def mk(s, backward):
            comm = comm_b if backward else comm_f
            src_slot = (N_DEV - 1) if s == 0 else s - 1
            return pltpu.make_async_remote_copy(
                src_ref=comm.at[src_slot],
                dst_ref=comm.at[s],
                send_sem=(send_b if backward else send_f).at[s],
                recv_sem=(recv_b if backward else recv_f).at[s],
                device_id=(left,) if backward else (right,),
                device_id_type=pl.DeviceIdType.MESH,
            )

        rdmas = [mk(0, False), mk(0, True)]
        rdmas[0].start()
        rdmas[1].start()

        pre_f = chunk_f(md(d - 2))
        pre_b = chunk_b(md(d + 2))

        for s in range(N_DEV - 1):
            rf, rb = rdmas[2 * s], rdmas[2 * s + 1]
            rf.wait_recv()
            rb.wait_recv()
            acc_f = comm_f[s].astype(jnp.float32) + pre_f
            acc_b = comm_b[s].astype(jnp.float32) + pre_b
            if s < N_DEV - 2:
                comm_f[s] = acc_f.astype(jnp.bfloat16)
                comm_b[s] = acc_b.astype(jnp.bfloat16)
                nf, nb = mk(s + 1, False), mk(s + 1, True)
                nf.start()
                nb.start()
                rdmas += [nf, nb]
                pre_f = chunk_f(md(d - s - 3))
                pre_b = chunk_b(md(d + s + 3))
            else:
                scale = sx_ref[0] * sw_ref[0]
                y_f = acc_f * scale
                y_b = acc_b * scale
                z_f = jnp.clip(y_f, -60.0, 60.0)
                z_b = jnp.clip(y_b, -60.0, 60.0)
                out_ref[:, :n_half] = y_f / (1.0 + jnp.exp(-z_f))
                out_ref[:, n_half:] = y_b / (1.0 + jnp.exp(-z_b))

        for r in rdmas:
            r.wait_send()

    return pl.pallas_call(
        body,
        out_shape=jax.ShapeDtypeStruct((m_chunk, n), jnp.float32),
        in_specs=[
            pl.BlockSpec(memory_space=pltpu.VMEM),
            pl.BlockSpec(memory_space=pltpu.VMEM),
            pl.BlockSpec(memory_space=pltpu.SMEM),
            pl.BlockSpec(memory_space=pltpu.SMEM),
        ],
        out_specs=pl.BlockSpec(memory_space=pltpu.VMEM),
        scratch_shapes=[
            pltpu.VMEM((N_DEV, m_chunk, n_half), jnp.bfloat16),
            pltpu.VMEM((N_DEV, m_chunk, n_half), jnp.bfloat16),
            pltpu.SemaphoreType.DMA((N_DEV - 1,)),
            pltpu.SemaphoreType.DMA((N_DEV - 1,)),
            pltpu.SemaphoreType.DMA((N_DEV - 1,)),
            pltpu.SemaphoreType.DMA((N_DEV - 1,)),
        ],
    )(xb, wb, sx, sw)


# device time: 130782 ns/iter; 1.6733x vs baseline; 1.1656x over previous
import jax
import jax.numpy as jnp
from jax import lax
from jax.experimental import pallas as pl
from jax.experimental.pallas import tpu as pltpu

N_DEV = 16


def kernel(x, w_mat, scale_x, scale_w):
    m, _ = x.shape
    _, n = w_mat.shape
    m_chunk = m // N_DEV
    n_half = n // 2

    xb = x.astype(jnp.bfloat16)
    wb = w_mat.astype(jnp.bfloat16)
    sx = scale_x.reshape(1).astype(jnp.float32)
    sw = scale_w.reshape(1).astype(jnp.float32)

    def body(
        x_ref,
        w_ref,
        sx_ref,
        sw_ref,
        out_ref,
        comm_f,
        comm_b,
        send_f,
        recv_f,
        send_b,
        recv_b,
    ):
        d = lax.axis_index("i")
        right = lax.rem(d + 1, N_DEV)
        left = lax.rem(d - 1 + N_DEV, N_DEV)

        def chunk_f(c):
            xc = x_ref[pl.ds(c * m_chunk, m_chunk), :]
            return jnp.dot(
                xc, w_ref[:, :n_half], preferred_element_type=jnp.float32
            )

        def chunk_b(c):
            xc = x_ref[pl.ds(c * m_chunk, m_chunk), :]
            return jnp.dot(
                xc, w_ref[:, n_half:], preferred_element_type=jnp.float32
            )

        def md(v):
            return lax.rem(v + 2 * N_DEV, N_DEV)

        comm_f[N_DEV - 1] = chunk_f(md(d - 1)).astype(jnp.bfloat16)
        comm_b[N_DEV - 1] = chunk_b(md(d + 1)).astype(jnp.bfloat16)

        def mk(s, backward):
            comm = comm_b if backward else comm_f
            src_slot = (N_DEV - 1) if s == 0 else s - 1
            return pltpu.make_async_remote_copy(
                src_ref=comm.at[src_slot],
                dst_ref=comm.at[s],
                send_sem=(send_b if backward else send_f).at[s],
                recv_sem=(recv_b if backward else recv_f).at[s],
                device_id=(left,) if backward else (right,),
                device_id_type=pl.DeviceIdType.MESH,
            )

        rdmas = [mk(0, False), mk(0, True)]
        rdmas[0].start()
        rdmas[1].start()

        pre_f = chunk_f(md(d - 2)).astype(jnp.bfloat16)
        pre_b = chunk_b(md(d + 2)).astype(jnp.bfloat16)

        for s in range(N_DEV - 1):
            rf, rb = rdmas[2 * s], rdmas[2 * s + 1]
            if s < N_DEV - 2:
                rf.wait_recv()
                comm_f[s] = comm_f[s] + pre_f
                nf = mk(s + 1, False)
                nf.start()
                rb.wait_recv()
                comm_b[s] = comm_b[s] + pre_b
                nb = mk(s + 1, True)
                nb.start()
                rdmas += [nf, nb]
                pre_f = chunk_f(md(d - s - 3)).astype(jnp.bfloat16)
                pre_b = chunk_b(md(d + s + 3)).astype(jnp.bfloat16)
            else:
                rf.wait_recv()
                rb.wait_recv()
                acc_f = comm_f[s].astype(jnp.float32) + pre_f.astype(
                    jnp.float32
                )
                acc_b = comm_b[s].astype(jnp.float32) + pre_b.astype(
                    jnp.float32
                )
                scale = sx_ref[0] * sw_ref[0]
                y_f = acc_f * scale
                y_b = acc_b * scale
                z_f = jnp.clip(y_f, -60.0, 60.0)
                z_b = jnp.clip(y_b, -60.0, 60.0)
                out_ref[:, :n_half] = y_f / (1.0 + jnp.exp(-z_f))
                out_ref[:, n_half:] = y_b / (1.0 + jnp.exp(-z_b))

        for r in rdmas:
            r.wait_send()

    return pl.pallas_call(
        body,
        out_shape=jax.ShapeDtypeStruct((m_chunk, n), jnp.float32),
        in_specs=[
            pl.BlockSpec(memory_space=pltpu.VMEM),
            pl.BlockSpec(memory_space=pltpu.VMEM),
            pl.BlockSpec(memory_space=pltpu.SMEM),
            pl.BlockSpec(memory_space=pltpu.SMEM),
        ],
        out_specs=pl.BlockSpec(memory_space=pltpu.VMEM),
        scratch_shapes=[
            pltpu.VMEM((N_DEV, m_chunk, n_half), jnp.bfloat16),
            pltpu.VMEM((N_DEV, m_chunk, n_half), jnp.bfloat16),
            pltpu.SemaphoreType.DMA((N_DEV - 1,)),
            pltpu.SemaphoreType.DMA((N_DEV - 1,)),
            pltpu.SemaphoreType.DMA((N_DEV - 1,)),
            pltpu.SemaphoreType.DMA((N_DEV - 1,)),
        ],
    )(xb, wb, sx, sw)


# device time: 104948 ns/iter; 2.0852x vs baseline; 1.2462x over previous
import jax
import jax.numpy as jnp
from jax import lax
from jax.experimental import pallas as pl
from jax.experimental.pallas import tpu as pltpu

N_DEV = 16


def kernel(x, w_mat, scale_x, scale_w):
    m, _ = x.shape
    _, n = w_mat.shape
    m_chunk = m // N_DEV
    n_half = n // 2

    xb = x.astype(jnp.bfloat16)
    wb = w_mat.astype(jnp.bfloat16)
    sx = scale_x.reshape(1).astype(jnp.float32)
    sw = scale_w.reshape(1).astype(jnp.float32)

    def body(
        x_ref,
        w_ref,
        sx_ref,
        sw_ref,
        out_ref,
        comm_f,
        comm_b,
        send_f,
        recv_f,
        send_b,
        recv_b,
    ):
        d = lax.axis_index("i")
        right = lax.rem(d + 1, N_DEV)
        left = lax.rem(d - 1 + N_DEV, N_DEV)

        def chunk_f(c):
            xc = x_ref[pl.ds(c * m_chunk, m_chunk), :]
            return jnp.dot(
                xc, w_ref[:, :n_half], preferred_element_type=jnp.float32
            )

        def chunk_b(c):
            xc = x_ref[pl.ds(c * m_chunk, m_chunk), :]
            return jnp.dot(
                xc, w_ref[:, n_half:], preferred_element_type=jnp.float32
            )

        def md(v):
            return lax.rem(v + 2 * N_DEV, N_DEV)

        comm_f[N_DEV - 1] = chunk_f(md(d - 1)).astype(jnp.bfloat16)
        comm_b[N_DEV - 1] = chunk_b(md(d + 1)).astype(jnp.bfloat16)

        n_sub = n_half // 2

        def mk(s, backward, j):
            comm = comm_b if backward else comm_f
            src_slot = (N_DEV - 1) if s == 0 else s - 1
            col = slice(j * n_sub, (j + 1) * n_sub)
            return pltpu.make_async_remote_copy(
                src_ref=comm.at[src_slot, :, col],
                dst_ref=comm.at[s, :, col],
                send_sem=(send_b if backward else send_f).at[s, j],
                recv_sem=(recv_b if backward else recv_f).at[s, j],
                device_id=(left,) if backward else (right,),
                device_id_type=pl.DeviceIdType.MESH,
            )

        rdmas = [mk(0, False, 0), mk(0, True, 0), mk(0, False, 1), mk(0, True, 1)]
        for r in rdmas:
            r.start()

        pre_f = chunk_f(md(d - 2)).astype(jnp.bfloat16)
        pre_b = chunk_b(md(d + 2)).astype(jnp.bfloat16)

        cols_a = slice(0, n_sub)
        cols_b2 = slice(n_sub, n_half)

        for s in range(N_DEV - 1):
            rfa, rba, rfb, rbb = rdmas[4 * s : 4 * s + 4]
            if s < N_DEV - 2:
                nxt = []
                for r, comm, pre, col, bwd in (
                    (rfa, comm_f, pre_f, cols_a, False),
                    (rba, comm_b, pre_b, cols_a, True),
                    (rfb, comm_f, pre_f, cols_b2, False),
                    (rbb, comm_b, pre_b, cols_b2, True),
                ):
                    r.wait_recv()
                    comm[s, :, col] = comm[s, :, col] + pre[:, col]
                    nr = mk(s + 1, bwd, 0 if col is cols_a else 1)
                    nr.start()
                    nxt.append(nr)
                rdmas += nxt
                pre_f = chunk_f(md(d - s - 3)).astype(jnp.bfloat16)
                pre_b = chunk_b(md(d + s + 3)).astype(jnp.bfloat16)
            else:
                for r in (rfa, rba, rfb, rbb):
                    r.wait_recv()
                acc_f = comm_f[s].astype(jnp.float32) + pre_f.astype(
                    jnp.float32
                )
                acc_b = comm_b[s].astype(jnp.float32) + pre_b.astype(
                    jnp.float32
                )
                scale = sx_ref[0] * sw_ref[0]
                y_f = acc_f * scale
                y_b = acc_b * scale
                z_f = jnp.clip(y_f, -60.0, 60.0)
                z_b = jnp.clip(y_b, -60.0, 60.0)
                out_ref[:, :n_half] = y_f / (1.0 + jnp.exp(-z_f))
                out_ref[:, n_half:] = y_b / (1.0 + jnp.exp(-z_b))

        for r in rdmas:
            r.wait_send()

    return pl.pallas_call(
        body,
        out_shape=jax.ShapeDtypeStruct((m_chunk, n), jnp.float32),
        in_specs=[
            pl.BlockSpec(memory_space=pltpu.VMEM),
            pl.BlockSpec(memory_space=pltpu.VMEM),
            pl.BlockSpec(memory_space=pltpu.SMEM),
            pl.BlockSpec(memory_space=pltpu.SMEM),
        ],
        out_specs=pl.BlockSpec(memory_space=pltpu.VMEM),
        scratch_shapes=[
            pltpu.VMEM((N_DEV, m_chunk, n_half), jnp.bfloat16),
            pltpu.VMEM((N_DEV, m_chunk, n_half), jnp.bfloat16),
            pltpu.SemaphoreType.DMA((N_DEV - 1, 2)),
            pltpu.SemaphoreType.DMA((N_DEV - 1, 2)),
            pltpu.SemaphoreType.DMA((N_DEV - 1, 2)),
            pltpu.SemaphoreType.DMA((N_DEV - 1, 2)),
        ],
    )(xb, wb, sx, sw)


# device time: 104918 ns/iter; 2.0858x vs baseline; 1.0003x over previous
import jax
import jax.numpy as jnp
from jax import lax
from jax.experimental import pallas as pl
from jax.experimental.pallas import tpu as pltpu

N_DEV = 16


def kernel(x, w_mat, scale_x, scale_w):
    m, _ = x.shape
    _, n = w_mat.shape
    m_chunk = m // N_DEV
    n_half = n // 2

    sx = scale_x.reshape(1).astype(jnp.float32)
    sw = scale_w.reshape(1).astype(jnp.float32)

    def body(
        x_ref,
        w_ref,
        sx_ref,
        sw_ref,
        out_ref,
        comm_f,
        comm_b,
        send_f,
        recv_f,
        send_b,
        recv_b,
    ):
        d = lax.axis_index("i")
        right = lax.rem(d + 1, N_DEV)
        left = lax.rem(d - 1 + N_DEV, N_DEV)

        def chunk_f(c):
            xc = x_ref[pl.ds(c * m_chunk, m_chunk), :]
            return jnp.dot(
                xc, w_ref[:, :n_half], preferred_element_type=jnp.float32
            )

        def chunk_b(c):
            xc = x_ref[pl.ds(c * m_chunk, m_chunk), :]
            return jnp.dot(
                xc, w_ref[:, n_half:], preferred_element_type=jnp.float32
            )

        def md(v):
            return lax.rem(v + 2 * N_DEV, N_DEV)

        comm_f[N_DEV - 1] = chunk_f(md(d - 1)).astype(jnp.bfloat16)
        comm_b[N_DEV - 1] = chunk_b(md(d + 1)).astype(jnp.bfloat16)

        n_sub = n_half // 2

        def mk(s, backward, j):
            comm = comm_b if backward else comm_f
            src_slot = (N_DEV - 1) if s == 0 else s - 1
            col = slice(j * n_sub, (j + 1) * n_sub)
            return pltpu.make_async_remote_copy(
                src_ref=comm.at[src_slot, :, col],
                dst_ref=comm.at[s, :, col],
                send_sem=(send_b if backward else send_f).at[s, j],
                recv_sem=(recv_b if backward else recv_f).at[s, j],
                device_id=(left,) if backward else (right,),
                device_id_type=pl.DeviceIdType.MESH,
            )

        rdmas = [mk(0, False, 0), mk(0, True, 0), mk(0, False, 1), mk(0, True, 1)]
        for r in rdmas:
            r.start()

        pre_f = chunk_f(md(d - 2)).astype(jnp.bfloat16)
        pre_b = chunk_b(md(d + 2)).astype(jnp.bfloat16)

        cols_a = slice(0, n_sub)
        cols_b2 = slice(n_sub, n_half)

        for s in range(N_DEV - 1):
            rfa, rba, rfb, rbb = rdmas[4 * s : 4 * s + 4]
            if s < N_DEV - 2:
                nxt = []
                for r, comm, pre, col, bwd in (
                    (rfa, comm_f, pre_f, cols_a, False),
                    (rba, comm_b, pre_b, cols_a, True),
                    (rfb, comm_f, pre_f, cols_b2, False),
                    (rbb, comm_b, pre_b, cols_b2, True),
                ):
                    r.wait_recv()
                    comm[s, :, col] = comm[s, :, col] + pre[:, col]
                    nr = mk(s + 1, bwd, 0 if col is cols_a else 1)
                    nr.start()
                    nxt.append(nr)
                rdmas += nxt
                pre_f = chunk_f(md(d - s - 3)).astype(jnp.bfloat16)
                pre_b = chunk_b(md(d + s + 3)).astype(jnp.bfloat16)
            else:
                for r in (rfa, rba, rfb, rbb):
                    r.wait_recv()
                acc_f = comm_f[s].astype(jnp.float32) + pre_f.astype(
                    jnp.float32
                )
                acc_b = comm_b[s].astype(jnp.float32) + pre_b.astype(
                    jnp.float32
                )
                scale = sx_ref[0] * sw_ref[0]
                y_f = acc_f * scale
                y_b = acc_b * scale
                z_f = jnp.clip(y_f, -60.0, 60.0)
                z_b = jnp.clip(y_b, -60.0, 60.0)
                out_ref[:, :n_half] = y_f / (1.0 + jnp.exp(-z_f))
                out_ref[:, n_half:] = y_b / (1.0 + jnp.exp(-z_b))

        for r in rdmas:
            r.wait_send()

    return pl.pallas_call(
        body,
        out_shape=jax.ShapeDtypeStruct((m_chunk, n), jnp.float32),
        in_specs=[
            pl.BlockSpec(memory_space=pltpu.VMEM),
            pl.BlockSpec(memory_space=pltpu.VMEM),
            pl.BlockSpec(memory_space=pltpu.SMEM),
            pl.BlockSpec(memory_space=pltpu.SMEM),
        ],
        out_specs=pl.BlockSpec(memory_space=pltpu.VMEM),
        scratch_shapes=[
            pltpu.VMEM((N_DEV, m_chunk, n_half), jnp.bfloat16),
            pltpu.VMEM((N_DEV, m_chunk, n_half), jnp.bfloat16),
            pltpu.SemaphoreType.DMA((N_DEV - 1, 2)),
            pltpu.SemaphoreType.DMA((N_DEV - 1, 2)),
            pltpu.SemaphoreType.DMA((N_DEV - 1, 2)),
            pltpu.SemaphoreType.DMA((N_DEV - 1, 2)),
        ],
    )(x, w_mat, sx, sw)


# device time: 98984 ns/iter; 2.2108x vs baseline; 1.0599x over previous
import jax
import jax.numpy as jnp
from jax import lax
from jax.experimental import pallas as pl
from jax.experimental.pallas import tpu as pltpu

N_DEV = 16


def kernel(x, w_mat, scale_x, scale_w):
    m, _ = x.shape
    _, n = w_mat.shape
    m_chunk = m // N_DEV
    n_half = n // 2

    sx = scale_x.reshape(1).astype(jnp.float32)
    sw = scale_w.reshape(1).astype(jnp.float32)

    def body(
        x_ref,
        w_ref,
        sx_ref,
        sw_ref,
        out_ref,
        comm_f,
        comm_b,
        send_f,
        recv_f,
        send_b,
        recv_b,
    ):
        d = lax.axis_index("i")
        right = lax.rem(d + 1, N_DEV)
        left = lax.rem(d - 1 + N_DEV, N_DEV)

        barrier_sem = pltpu.get_barrier_semaphore()
        for nbr in (left, right):
            pl.semaphore_signal(
                barrier_sem,
                inc=1,
                device_id=(nbr,),
                device_id_type=pl.DeviceIdType.MESH,
            )
        pl.semaphore_wait(barrier_sem, 2)

        def chunk_f(c):
            xc = x_ref[pl.ds(c * m_chunk, m_chunk), :]
            return jnp.dot(
                xc, w_ref[:, :n_half], preferred_element_type=jnp.float32
            )

        def chunk_b(c):
            xc = x_ref[pl.ds(c * m_chunk, m_chunk), :]
            return jnp.dot(
                xc, w_ref[:, n_half:], preferred_element_type=jnp.float32
            )

        def md(v):
            return lax.rem(v + 2 * N_DEV, N_DEV)

        comm_f[N_DEV - 1] = chunk_f(md(d - 1)).astype(jnp.bfloat16)
        comm_b[N_DEV - 1] = chunk_b(md(d + 1)).astype(jnp.bfloat16)

        m_sub = m_chunk // 2

        def mk(s, backward, j):
            comm = comm_b if backward else comm_f
            src_slot = (N_DEV - 1) if s == 0 else s - 1
            rows = slice(j * m_sub, (j + 1) * m_sub)
            return pltpu.make_async_remote_copy(
                src_ref=comm.at[src_slot, rows, :],
                dst_ref=comm.at[s, rows, :],
                send_sem=(send_b if backward else send_f).at[s, j],
                recv_sem=(recv_b if backward else recv_f).at[s, j],
                device_id=(left,) if backward else (right,),
                device_id_type=pl.DeviceIdType.MESH,
            )

        rdmas = [mk(0, False, 0), mk(0, True, 0), mk(0, False, 1), mk(0, True, 1)]
        for r in rdmas:
            r.start()

        pre_f = chunk_f(md(d - 2)).astype(jnp.bfloat16)
        pre_b = chunk_b(md(d + 2)).astype(jnp.bfloat16)

        rows_a = slice(0, m_sub)
        rows_b2 = slice(m_sub, m_chunk)

        scale = sx_ref[0] * sw_ref[0]

        def epilogue(acc):
            y = acc * scale
            z = jnp.clip(y, -60.0, 60.0)
            return y / (1.0 + jnp.exp(-z))

        for s in range(N_DEV - 1):
            rfa, rba, rfb, rbb = rdmas[4 * s : 4 * s + 4]
            pieces = (
                (rfa, comm_f, rows_a, False),
                (rba, comm_b, rows_a, True),
                (rfb, comm_f, rows_b2, False),
                (rbb, comm_b, rows_b2, True),
            )
            if s < N_DEV - 2:
                nxt = []
                for r, comm, rows, bwd in pieces:
                    pre = pre_b if bwd else pre_f
                    r.wait_recv()
                    comm[s, rows, :] = comm[s, rows, :] + pre[rows, :]
                    nr = mk(s + 1, bwd, 0 if rows is rows_a else 1)
                    nr.start()
                    nxt.append(nr)
                rdmas += nxt
                pre_f = chunk_f(md(d - s - 3)).astype(jnp.bfloat16)
                pre_b = chunk_b(md(d + s + 3)).astype(jnp.bfloat16)
            else:
                for r, comm, rows, bwd in pieces:
                    pre = pre_b if bwd else pre_f
                    r.wait_recv()
                    acc = comm[s, rows, :].astype(jnp.float32) + pre[
                        rows, :
                    ].astype(jnp.float32)
                    cols = slice(n_half, n) if bwd else slice(0, n_half)
                    out_ref[rows, cols] = epilogue(acc)

        for r in rdmas:
            r.wait_send()

    return pl.pallas_call(
        body,
        out_shape=jax.ShapeDtypeStruct((m_chunk, n), jnp.float32),
        in_specs=[
            pl.BlockSpec(memory_space=pltpu.VMEM),
            pl.BlockSpec(memory_space=pltpu.VMEM),
            pl.BlockSpec(memory_space=pltpu.SMEM),
            pl.BlockSpec(memory_space=pltpu.SMEM),
        ],
        out_specs=pl.BlockSpec(memory_space=pltpu.VMEM),
        scratch_shapes=[
            pltpu.VMEM((N_DEV, m_chunk, n_half), jnp.bfloat16),
            pltpu.VMEM((N_DEV, m_chunk, n_half), jnp.bfloat16),
            pltpu.SemaphoreType.DMA((N_DEV - 1, 2)),
            pltpu.SemaphoreType.DMA((N_DEV - 1, 2)),
            pltpu.SemaphoreType.DMA((N_DEV - 1, 2)),
            pltpu.SemaphoreType.DMA((N_DEV - 1, 2)),
        ],
        compiler_params=pltpu.CompilerParams(collective_id=0),
    )(x, w_mat, sx, sw)
